# Initial kernel scaffold; baseline (speedup 1.0000x reference)
import jax
import jax.numpy as jnp
from jax.experimental import pallas as pl


def kernel(x, edge_index, W_in, b_in, W_layers, b_layers, gamma, beta, W_out, b_out):
    raise NotImplementedError("write your pallas kernel here")


# speedup vs baseline: 10.3348x; 10.3348x over previous
import functools

import jax
import jax.numpy as jnp
from jax import lax
from jax.experimental import pallas as pl
from jax.experimental.pallas import tpu as pltpu
from jax.experimental.pallas import tpu_sc as plsc

N = 10000
E = 320000
D = 128
H = 128
C = 40
NLAYERS = 3
EPS = 1e-5

NC = 2
NS = 16
NW = NC * NS
CH = 128
NCHUNK = E // CH
MAXJ = (NCHUNK + NW - 1) // NW
RPT = 624
TAIL = N - RPT * NS
TAIL_OFF = RPT * NS
DEGW = 128


def _striped_copy(s, src_ref, dst_ref):
    pltpu.sync_copy(src_ref.at[pl.ds(s * RPT, RPT)],
                    dst_ref.at[pl.ds(s * RPT, RPT)])

    @pl.when(s == NS - 1)
    def _():
        pltpu.sync_copy(src_ref.at[pl.ds(TAIL_OFF, TAIL)],
                        dst_ref.at[pl.ds(TAIL_OFF, TAIL)])


def _deg_body(dst_hbm, ones_hbm, zeros_hbm, out_hbm, didx, ones_v, acc):
    c = lax.axis_index("c")
    s = lax.axis_index("s")
    w = s * NC + c
    pltpu.sync_copy(ones_hbm, ones_v)
    _striped_copy(s, zeros_hbm, acc)
    plsc.subcore_barrier()

    def body(j, carry):
        k = j * NW + w

        @pl.when(k < NCHUNK)
        def _():
            pltpu.sync_copy(dst_hbm.at[pl.ds(k * CH, CH)], didx)
            pltpu.sync_copy(ones_v, acc.at[didx], add=True)

        return carry

    lax.fori_loop(0, MAXJ, body, 0)
    plsc.subcore_barrier()
    _striped_copy(s, acc, out_hbm.at[c])


def _spmm_body(hp_hbm, src_hbm, dst_hbm, zeros_hbm, out_hbm,
               sidx, didx, rows, acc, sem):
    c = lax.axis_index("c")
    s = lax.axis_index("s")
    w = s * NC + c
    _striped_copy(s, zeros_hbm, acc)
    plsc.subcore_barrier()

    def body(j, carry):
        k = j * NW + w

        @pl.when(k < NCHUNK)
        def _():
            off = k * CH
            pltpu.sync_copy(src_hbm.at[pl.ds(off, CH)], sidx)
            pltpu.sync_copy(dst_hbm.at[pl.ds(off, CH)], didx)
            pltpu.async_copy(hp_hbm.at[sidx], rows, sem).wait()
            pltpu.sync_copy(rows, acc.at[didx], add=True)

        return carry

    lax.fori_loop(0, MAXJ, body, 0)
    plsc.subcore_barrier()
    _striped_copy(s, acc, out_hbm.at[c])


@functools.cache
def _sc_kernels():
    mesh = plsc.VectorSubcoreMesh(
        core_axis_name="c", subcore_axis_name="s",
        num_cores=NC, num_subcores=NS,
    )
    deg = pl.kernel(
        _deg_body,
        out_type=jax.ShapeDtypeStruct((NC, N, DEGW), jnp.float32),
        mesh=mesh,
        scratch_types=[
            pltpu.VMEM((CH,), jnp.int32),
            pltpu.VMEM((CH, DEGW), jnp.float32),
            pltpu.VMEM_SHARED((N, DEGW), jnp.float32),
        ],
    )
    spmm = pl.kernel(
        _spmm_body,
        out_type=jax.ShapeDtypeStruct((NC, N, H), jnp.float32),
        mesh=mesh,
        scratch_types=[
            pltpu.VMEM((CH,), jnp.int32),
            pltpu.VMEM((CH,), jnp.int32),
            pltpu.VMEM((CH, H), jnp.float32),
            pltpu.VMEM_SHARED((N, H), jnp.float32),
            pltpu.SemaphoreType.DMA,
        ],
    )
    return deg, spmm


def _tc_in_body(degp_ref, x_ref, w_ref, b_ref, h0_ref, hp0_ref, d_ref):
    deg = degp_ref[0] + degp_ref[1]
    d = lax.rsqrt(jnp.maximum(deg, 1.0))
    d_ref[...] = d
    h0 = jnp.dot(x_ref[...], w_ref[...],
                 preferred_element_type=jnp.float32) + b_ref[...]
    h0_ref[...] = h0
    hp0_ref[...] = h0 * d[:, 0:1]


def _tc_layer_core(aggp_ref, d_ref, hres_ref, w_ref, b_ref, g_ref, be_ref):
    d = d_ref[:, 0:1]
    agg = (aggp_ref[0] + aggp_ref[1]) * d
    t = jnp.dot(agg, w_ref[...],
                preferred_element_type=jnp.float32) + b_ref[...]
    mean = jnp.mean(t, axis=0, keepdims=True)
    ctr = t - mean
    var = jnp.mean(ctr * ctr, axis=0, keepdims=True)
    tn = ctr * lax.rsqrt(var + EPS) * g_ref[...] + be_ref[...]
    h = jnp.maximum(tn, 0.0) + hres_ref[...]
    return h, d


def _tc_layer_body(aggp_ref, d_ref, hres_ref, w_ref, b_ref, g_ref, be_ref,
                   h_ref, hp_ref):
    h, d = _tc_layer_core(aggp_ref, d_ref, hres_ref, w_ref, b_ref, g_ref,
                          be_ref)
    h_ref[...] = h
    hp_ref[...] = h * d


def _tc_last_body(aggp_ref, d_ref, hres_ref, w_ref, b_ref, g_ref, be_ref,
                  wo_ref, bo_ref, out_ref):
    h, _ = _tc_layer_core(aggp_ref, d_ref, hres_ref, w_ref, b_ref, g_ref,
                          be_ref)
    out_ref[...] = jnp.dot(h, wo_ref[...],
                           preferred_element_type=jnp.float32) + bo_ref[...]


_tc_in = pl.pallas_call(
    _tc_in_body,
    out_shape=[
        jax.ShapeDtypeStruct((N, H), jnp.float32),
        jax.ShapeDtypeStruct((N, H), jnp.float32),
        jax.ShapeDtypeStruct((N, DEGW), jnp.float32),
    ],
)

_tc_layer = pl.pallas_call(
    _tc_layer_body,
    out_shape=[
        jax.ShapeDtypeStruct((N, H), jnp.float32),
        jax.ShapeDtypeStruct((N, H), jnp.float32),
    ],
)

_tc_last = pl.pallas_call(
    _tc_last_body,
    out_shape=jax.ShapeDtypeStruct((N, C), jnp.float32),
)


def kernel(x, edge_index, W_in, b_in, W_layers, b_layers, gamma, beta,
           W_out, b_out):
    ei = edge_index.astype(jnp.int32)
    src = ei[0]
    dst = ei[1]
    zeros_h = jnp.zeros((N, H), jnp.float32)
    zeros_d = jnp.zeros((N, DEGW), jnp.float32)
    ones_d = jnp.ones((CH, DEGW), jnp.float32)

    deg_kernel, spmm_kernel = _sc_kernels()
    degp = deg_kernel(dst, ones_d, zeros_d)
    h, hp, d = _tc_in(degp, x, W_in, b_in.reshape(1, H))
    for i in range(NLAYERS):
        aggp = spmm_kernel(hp, src, dst, zeros_h)
        if i + 1 < NLAYERS:
            h, hp = _tc_layer(
                aggp, d, h, W_layers[i], b_layers[i].reshape(1, H),
                gamma[i].reshape(1, H), beta[i].reshape(1, H)
            )
        else:
            out = _tc_last(
                aggp, d, h, W_layers[i], b_layers[i].reshape(1, H),
                gamma[i].reshape(1, H), beta[i].reshape(1, H),
                W_out, b_out.reshape(1, C)
            )
    return out


# speedup vs baseline: 14.9068x; 1.4424x over previous
import functools

import jax
import jax.numpy as jnp
from jax import lax
from jax.experimental import pallas as pl
from jax.experimental.pallas import tpu as pltpu
from jax.experimental.pallas import tpu_sc as plsc

N = 10000
E = 320000
D = 128
H = 128
C = 40
NLAYERS = 3
EPS = 1e-5

NC = 2
NS = 16
NW = NC * NS
CH = 128
NCHUNK = E // CH
MAXJ = (NCHUNK + NW - 1) // NW
RPT = 624
TAIL = N - RPT * NS
TAIL_OFF = RPT * NS
DEGW = 128


def _striped_copy(s, src_ref, dst_ref):
    pltpu.sync_copy(src_ref.at[pl.ds(s * RPT, RPT)],
                    dst_ref.at[pl.ds(s * RPT, RPT)])

    @pl.when(s == NS - 1)
    def _():
        pltpu.sync_copy(src_ref.at[pl.ds(TAIL_OFF, TAIL)],
                        dst_ref.at[pl.ds(TAIL_OFF, TAIL)])


def _deg_body(dst_hbm, ones_hbm, zeros_hbm, out_hbm, didx, ones_v, acc):
    c = lax.axis_index("c")
    s = lax.axis_index("s")
    w = s * NC + c
    pltpu.sync_copy(ones_hbm, ones_v)
    _striped_copy(s, zeros_hbm, acc)
    plsc.subcore_barrier()

    def body(j, carry):
        k = j * NW + w

        @pl.when(k < NCHUNK)
        def _():
            pltpu.sync_copy(dst_hbm.at[pl.ds(k * CH, CH)], didx)
            pltpu.sync_copy(ones_v, acc.at[didx], add=True)

        return carry

    lax.fori_loop(0, MAXJ, body, 0)
    plsc.subcore_barrier()
    _striped_copy(s, acc, out_hbm.at[c])


def _spmm_body(hp_hbm, src_hbm, dst_hbm, zeros_hbm, out_hbm,
               sidx0, didx0, rows0, sidx1, didx1, rows1, acc, sem0, sem1):
    c = lax.axis_index("c")
    s = lax.axis_index("s")
    w = s * NC + c
    _striped_copy(s, zeros_hbm, acc)
    plsc.subcore_barrier()

    bufs = ((sidx0, didx0, rows0, sem0), (sidx1, didx1, rows1, sem1))

    def start(j, b):
        sidx, didx, rows, sem = bufs[b]

        @pl.when(j * NW + w < NCHUNK)
        def _():
            off = (j * NW + w) * CH
            pltpu.sync_copy(src_hbm.at[pl.ds(off, CH)], sidx)
            pltpu.sync_copy(dst_hbm.at[pl.ds(off, CH)], didx)
            pltpu.async_copy(hp_hbm.at[sidx], rows, sem)

    def drain(j, b):
        sidx, didx, rows, sem = bufs[b]

        @pl.when(j * NW + w < NCHUNK)
        def _():
            pltpu.make_async_copy(hp_hbm.at[sidx], rows, sem).wait()
            pltpu.sync_copy(rows, acc.at[didx], add=True)

    start(0, 0)

    def body(i, carry):
        j0 = 2 * i
        start(j0 + 1, 1)
        drain(j0, 0)
        start(j0 + 2, 0)
        drain(j0 + 1, 1)
        return carry

    lax.fori_loop(0, (MAXJ + 1) // 2, body, 0)
    plsc.subcore_barrier()
    _striped_copy(s, acc, out_hbm.at[c])


@functools.cache
def _sc_kernels():
    mesh = plsc.VectorSubcoreMesh(
        core_axis_name="c", subcore_axis_name="s",
        num_cores=NC, num_subcores=NS,
    )
    deg = pl.kernel(
        _deg_body,
        out_type=jax.ShapeDtypeStruct((NC, N, DEGW), jnp.float32),
        mesh=mesh,
        scratch_types=[
            pltpu.VMEM((CH,), jnp.int32),
            pltpu.VMEM((CH, DEGW), jnp.float32),
            pltpu.VMEM_SHARED((N, DEGW), jnp.float32),
        ],
    )
    spmm = pl.kernel(
        _spmm_body,
        out_type=jax.ShapeDtypeStruct((NC, N, H), jnp.float32),
        mesh=mesh,
        scratch_types=[
            pltpu.VMEM((CH,), jnp.int32),
            pltpu.VMEM((CH,), jnp.int32),
            pltpu.VMEM((CH, H), jnp.float32),
            pltpu.VMEM((CH,), jnp.int32),
            pltpu.VMEM((CH,), jnp.int32),
            pltpu.VMEM((CH, H), jnp.float32),
            pltpu.VMEM_SHARED((N, H), jnp.float32),
            pltpu.SemaphoreType.DMA,
            pltpu.SemaphoreType.DMA,
        ],
    )
    return deg, spmm


def _tc_in_body(degp_ref, x_ref, w_ref, b_ref, h0_ref, hp0_ref, d_ref):
    deg = degp_ref[0] + degp_ref[1]
    d = lax.rsqrt(jnp.maximum(deg, 1.0))
    d_ref[...] = d
    h0 = jnp.dot(x_ref[...], w_ref[...],
                 preferred_element_type=jnp.float32) + b_ref[...]
    h0_ref[...] = h0
    hp0_ref[...] = h0 * d[:, 0:1]


def _tc_layer_core(aggp_ref, d_ref, hres_ref, w_ref, b_ref, g_ref, be_ref):
    d = d_ref[:, 0:1]
    agg = (aggp_ref[0] + aggp_ref[1]) * d
    t = jnp.dot(agg, w_ref[...],
                preferred_element_type=jnp.float32) + b_ref[...]
    mean = jnp.mean(t, axis=0, keepdims=True)
    ctr = t - mean
    var = jnp.mean(ctr * ctr, axis=0, keepdims=True)
    tn = ctr * lax.rsqrt(var + EPS) * g_ref[...] + be_ref[...]
    h = jnp.maximum(tn, 0.0) + hres_ref[...]
    return h, d


def _tc_layer_body(aggp_ref, d_ref, hres_ref, w_ref, b_ref, g_ref, be_ref,
                   h_ref, hp_ref):
    h, d = _tc_layer_core(aggp_ref, d_ref, hres_ref, w_ref, b_ref, g_ref,
                          be_ref)
    h_ref[...] = h
    hp_ref[...] = h * d


def _tc_last_body(aggp_ref, d_ref, hres_ref, w_ref, b_ref, g_ref, be_ref,
                  wo_ref, bo_ref, out_ref):
    h, _ = _tc_layer_core(aggp_ref, d_ref, hres_ref, w_ref, b_ref, g_ref,
                          be_ref)
    out_ref[...] = jnp.dot(h, wo_ref[...],
                           preferred_element_type=jnp.float32) + bo_ref[...]


_tc_in = pl.pallas_call(
    _tc_in_body,
    out_shape=[
        jax.ShapeDtypeStruct((N, H), jnp.float32),
        jax.ShapeDtypeStruct((N, H), jnp.float32),
        jax.ShapeDtypeStruct((N, DEGW), jnp.float32),
    ],
)

_tc_layer = pl.pallas_call(
    _tc_layer_body,
    out_shape=[
        jax.ShapeDtypeStruct((N, H), jnp.float32),
        jax.ShapeDtypeStruct((N, H), jnp.float32),
    ],
)

_tc_last = pl.pallas_call(
    _tc_last_body,
    out_shape=jax.ShapeDtypeStruct((N, C), jnp.float32),
)


def kernel(x, edge_index, W_in, b_in, W_layers, b_layers, gamma, beta,
           W_out, b_out):
    ei = edge_index.astype(jnp.int32)
    src = ei[0]
    dst = ei[1]
    zeros_h = jnp.zeros((N, H), jnp.float32)
    zeros_d = jnp.zeros((N, DEGW), jnp.float32)
    ones_d = jnp.ones((CH, DEGW), jnp.float32)

    deg_kernel, spmm_kernel = _sc_kernels()
    degp = deg_kernel(dst, ones_d, zeros_d)
    h, hp, d = _tc_in(degp, x, W_in, b_in.reshape(1, H))
    for i in range(NLAYERS):
        aggp = spmm_kernel(hp, src, dst, zeros_h)
        if i + 1 < NLAYERS:
            h, hp = _tc_layer(
                aggp, d, h, W_layers[i], b_layers[i].reshape(1, H),
                gamma[i].reshape(1, H), beta[i].reshape(1, H)
            )
        else:
            out = _tc_last(
                aggp, d, h, W_layers[i], b_layers[i].reshape(1, H),
                gamma[i].reshape(1, H), beta[i].reshape(1, H),
                W_out, b_out.reshape(1, C)
            )
    return out


# speedup vs baseline: 18.3042x; 1.2279x over previous
import functools

import jax
import jax.numpy as jnp
from jax import lax
from jax.experimental import pallas as pl
from jax.experimental.pallas import tpu as pltpu
from jax.experimental.pallas import tpu_sc as plsc

N = 10000
E = 320000
D = 128
H = 128
C = 40
NLAYERS = 3
EPS = 1e-5

NC = 2
NS = 16
NW = NC * NS
CH = 128
NCHUNK = E // CH
NB = 8
NBLK = 10
CPW = NB * NBLK
NCHUNK_PAD = NW * CPW
RPT = 624
TAIL = N - RPT * NS
TAIL_OFF = RPT * NS
DEGW = 128


def _striped_copy(s, src_ref, dst_ref):
    pltpu.sync_copy(src_ref.at[pl.ds(s * RPT, RPT)],
                    dst_ref.at[pl.ds(s * RPT, RPT)])

    @pl.when(s == NS - 1)
    def _():
        pltpu.sync_copy(src_ref.at[pl.ds(TAIL_OFF, TAIL)],
                        dst_ref.at[pl.ds(TAIL_OFF, TAIL)])


def _deg_body(dst2_hbm, ones_hbm, zeros_hbm, out_hbm, dblk, ones_v, acc):
    c = lax.axis_index("c")
    s = lax.axis_index("s")
    w = s * NC + c
    base = w * CPW
    pltpu.sync_copy(ones_hbm, ones_v)
    _striped_copy(s, zeros_hbm, acc)
    plsc.subcore_barrier()

    def blk_body(blk, carry):
        off = base + blk * NB

        @pl.when(off < NCHUNK)
        def _():
            pltpu.sync_copy(dst2_hbm.at[pl.ds(off, NB)], dblk)
            for b in range(NB):
                @pl.when(off + b < NCHUNK)
                def _b():
                    pltpu.sync_copy(ones_v, acc.at[dblk.at[b]], add=True)

        return carry

    lax.fori_loop(0, NBLK, blk_body, 0)
    plsc.subcore_barrier()
    _striped_copy(s, acc, out_hbm.at[c])


def _spmm_body(hp_hbm, src2_hbm, dst2_hbm, zeros_hbm, out_hbm,
               sblk0, dblk0, sblk1, dblk1, rows0, rows1, acc, sem0, sem1):
    c = lax.axis_index("c")
    s = lax.axis_index("s")
    w = s * NC + c
    base = w * CPW
    _striped_copy(s, zeros_hbm, acc)
    plsc.subcore_barrier()

    iblk = ((sblk0, dblk0), (sblk1, dblk1))
    rbuf = ((rows0, sem0), (rows1, sem1))

    def load_blk(blk, p):
        sb, db = iblk[p]
        off = base + blk * NB

        @pl.when(off < NCHUNK)
        def _():
            pltpu.sync_copy(src2_hbm.at[pl.ds(off, NB)], sb)
            pltpu.sync_copy(dst2_hbm.at[pl.ds(off, NB)], db)

    def start(t, p, b):
        sb, _ = iblk[p]
        rows, sem = rbuf[b % 2]

        @pl.when((base + t < NCHUNK) & (t < CPW))
        def _():
            pltpu.async_copy(hp_hbm.at[sb.at[b]], rows, sem)

    def proc(blk, p, pn):
        sb, db = iblk[p]
        for b in range(NB):
            t = blk * NB + b
            if b + 1 < NB:
                start(t + 1, p, b + 1)
            else:
                start(t + 1, pn, 0)
            rows, sem = rbuf[b % 2]

            @pl.when(base + t < NCHUNK)
            def _():
                pltpu.make_async_copy(hp_hbm.at[sb.at[b]], rows, sem).wait()
                pltpu.sync_copy(rows, acc.at[db.at[b]], add=True)

    load_blk(0, 0)
    start(0, 0, 0)

    def body(i, carry):
        blk0 = 2 * i
        load_blk(blk0 + 1, 1)
        proc(blk0, 0, 1)
        load_blk(blk0 + 2, 0)
        proc(blk0 + 1, 1, 0)
        return carry

    lax.fori_loop(0, NBLK // 2, body, 0)
    plsc.subcore_barrier()
    _striped_copy(s, acc, out_hbm.at[c])


@functools.cache
def _sc_kernels():
    mesh = plsc.VectorSubcoreMesh(
        core_axis_name="c", subcore_axis_name="s",
        num_cores=NC, num_subcores=NS,
    )
    deg = pl.kernel(
        _deg_body,
        out_type=jax.ShapeDtypeStruct((NC, N, DEGW), jnp.float32),
        mesh=mesh,
        scratch_types=[
            pltpu.VMEM((NB, CH), jnp.int32),
            pltpu.VMEM((CH, DEGW), jnp.float32),
            pltpu.VMEM_SHARED((N, DEGW), jnp.float32),
        ],
    )
    spmm = pl.kernel(
        _spmm_body,
        out_type=jax.ShapeDtypeStruct((NC, N, H), jnp.float32),
        mesh=mesh,
        scratch_types=[
            pltpu.VMEM((NB, CH), jnp.int32),
            pltpu.VMEM((NB, CH), jnp.int32),
            pltpu.VMEM((NB, CH), jnp.int32),
            pltpu.VMEM((NB, CH), jnp.int32),
            pltpu.VMEM((CH, H), jnp.float32),
            pltpu.VMEM((CH, H), jnp.float32),
            pltpu.VMEM_SHARED((N, H), jnp.float32),
            pltpu.SemaphoreType.DMA,
            pltpu.SemaphoreType.DMA,
        ],
    )
    return deg, spmm


def _tc_in_body(degp_ref, x_ref, w_ref, b_ref, h0_ref, hp0_ref, d_ref):
    deg = degp_ref[0] + degp_ref[1]
    d = lax.rsqrt(jnp.maximum(deg, 1.0))
    d_ref[...] = d
    h0 = jnp.dot(x_ref[...], w_ref[...],
                 preferred_element_type=jnp.float32) + b_ref[...]
    h0_ref[...] = h0
    hp0_ref[...] = h0 * d[:, 0:1]


def _tc_layer_core(aggp_ref, d_ref, hres_ref, w_ref, b_ref, g_ref, be_ref):
    d = d_ref[:, 0:1]
    agg = (aggp_ref[0] + aggp_ref[1]) * d
    t = jnp.dot(agg, w_ref[...],
                preferred_element_type=jnp.float32) + b_ref[...]
    mean = jnp.mean(t, axis=0, keepdims=True)
    ctr = t - mean
    var = jnp.mean(ctr * ctr, axis=0, keepdims=True)
    tn = ctr * lax.rsqrt(var + EPS) * g_ref[...] + be_ref[...]
    h = jnp.maximum(tn, 0.0) + hres_ref[...]
    return h, d


def _tc_layer_body(aggp_ref, d_ref, hres_ref, w_ref, b_ref, g_ref, be_ref,
                   h_ref, hp_ref):
    h, d = _tc_layer_core(aggp_ref, d_ref, hres_ref, w_ref, b_ref, g_ref,
                          be_ref)
    h_ref[...] = h
    hp_ref[...] = h * d


def _tc_last_body(aggp_ref, d_ref, hres_ref, w_ref, b_ref, g_ref, be_ref,
                  wo_ref, bo_ref, out_ref):
    h, _ = _tc_layer_core(aggp_ref, d_ref, hres_ref, w_ref, b_ref, g_ref,
                          be_ref)
    out_ref[...] = jnp.dot(h, wo_ref[...],
                           preferred_element_type=jnp.float32) + bo_ref[...]


_tc_in = pl.pallas_call(
    _tc_in_body,
    out_shape=[
        jax.ShapeDtypeStruct((N, H), jnp.float32),
        jax.ShapeDtypeStruct((N, H), jnp.float32),
        jax.ShapeDtypeStruct((N, DEGW), jnp.float32),
    ],
)

_tc_layer = pl.pallas_call(
    _tc_layer_body,
    out_shape=[
        jax.ShapeDtypeStruct((N, H), jnp.float32),
        jax.ShapeDtypeStruct((N, H), jnp.float32),
    ],
)

_tc_last = pl.pallas_call(
    _tc_last_body,
    out_shape=jax.ShapeDtypeStruct((N, C), jnp.float32),
)


def kernel(x, edge_index, W_in, b_in, W_layers, b_layers, gamma, beta,
           W_out, b_out):
    ei = edge_index.astype(jnp.int32)
    pad = NCHUNK_PAD * CH - E
    src2 = jnp.pad(ei[0], (0, pad)).reshape(NCHUNK_PAD, CH)
    dst2 = jnp.pad(ei[1], (0, pad)).reshape(NCHUNK_PAD, CH)
    zeros_h = jnp.zeros((N, H), jnp.float32)
    zeros_d = jnp.zeros((N, DEGW), jnp.float32)
    ones_d = jnp.ones((CH, DEGW), jnp.float32)

    deg_kernel, spmm_kernel = _sc_kernels()
    degp = deg_kernel(dst2, ones_d, zeros_d)
    h, hp, d = _tc_in(degp, x, W_in, b_in.reshape(1, H))
    for i in range(NLAYERS):
        aggp = spmm_kernel(hp, src2, dst2, zeros_h)
        if i + 1 < NLAYERS:
            h, hp = _tc_layer(
                aggp, d, h, W_layers[i], b_layers[i].reshape(1, H),
                gamma[i].reshape(1, H), beta[i].reshape(1, H)
            )
        else:
            out = _tc_last(
                aggp, d, h, W_layers[i], b_layers[i].reshape(1, H),
                gamma[i].reshape(1, H), beta[i].reshape(1, H),
                W_out, b_out.reshape(1, C)
            )
    return out


# speedup vs baseline: 18.6803x; 1.0205x over previous
import functools

import jax
import jax.numpy as jnp
from jax import lax
from jax.experimental import pallas as pl
from jax.experimental.pallas import tpu as pltpu
from jax.experimental.pallas import tpu_sc as plsc

N = 10000
E = 320000
D = 128
H = 128
C = 40
NLAYERS = 3
EPS = 1e-5

NC = 2
NS = 16
NW = NC * NS
CH = 64
NCHUNK = E // CH
NB = 8
NBLK = 20
CPW = NB * NBLK
NCHUNK_PAD = NW * CPW
RPT = 624
TAIL = N - RPT * NS
TAIL_OFF = RPT * NS
DEGW = 128


def _striped_copy(s, src_ref, dst_ref):
    pltpu.sync_copy(src_ref.at[pl.ds(s * RPT, RPT)],
                    dst_ref.at[pl.ds(s * RPT, RPT)])

    @pl.when(s == NS - 1)
    def _():
        pltpu.sync_copy(src_ref.at[pl.ds(TAIL_OFF, TAIL)],
                        dst_ref.at[pl.ds(TAIL_OFF, TAIL)])


def _deg_body(dst2_hbm, ones_hbm, zeros_hbm, out_hbm, dblk, ones_v, acc):
    c = lax.axis_index("c")
    s = lax.axis_index("s")
    w = s * NC + c
    base = w * CPW
    pltpu.sync_copy(ones_hbm, ones_v)
    _striped_copy(s, zeros_hbm, acc)
    plsc.subcore_barrier()

    def blk_body(blk, carry):
        off = base + blk * NB

        @pl.when(off < NCHUNK)
        def _():
            pltpu.sync_copy(dst2_hbm.at[pl.ds(off, NB)], dblk)
            for b in range(NB):
                @pl.when(off + b < NCHUNK)
                def _b():
                    pltpu.sync_copy(ones_v, acc.at[dblk.at[b]], add=True)

        return carry

    lax.fori_loop(0, NBLK, blk_body, 0)
    plsc.subcore_barrier()
    _striped_copy(s, acc, out_hbm.at[c])


NRB = 4


def _spmm_body(hp_hbm, src2_hbm, dst2_hbm, zeros_hbm, out_hbm,
               sblk0, dblk0, sblk1, dblk1,
               rows0, rows1, rows2, rows3, acc, sem0, sem1, sem2, sem3):
    c = lax.axis_index("c")
    s = lax.axis_index("s")
    w = s * NC + c
    base = w * CPW
    _striped_copy(s, zeros_hbm, acc)
    plsc.subcore_barrier()

    iblk = ((sblk0, dblk0), (sblk1, dblk1))
    rbuf = ((rows0, sem0), (rows1, sem1), (rows2, sem2), (rows3, sem3))

    def load_blk(blk, p):
        sb, db = iblk[p]
        off = base + blk * NB

        @pl.when(off < NCHUNK)
        def _():
            pltpu.sync_copy(src2_hbm.at[pl.ds(off, NB)], sb)
            pltpu.sync_copy(dst2_hbm.at[pl.ds(off, NB)], db)

    def start(t, p, b):
        sb, _ = iblk[p]
        rows, sem = rbuf[b % NRB]

        @pl.when((base + t < NCHUNK) & (t < CPW))
        def _():
            pltpu.async_copy(hp_hbm.at[sb.at[b]], rows, sem)

    def proc(blk, p, pn):
        sb, db = iblk[p]
        for b in range(NB):
            t = blk * NB + b
            bn = b + NRB - 1
            if bn < NB:
                start(t + NRB - 1, p, bn)
            else:
                start(t + NRB - 1, pn, bn - NB)
            rows, sem = rbuf[b % NRB]

            @pl.when(base + t < NCHUNK)
            def _():
                pltpu.make_async_copy(hp_hbm.at[sb.at[b]], rows, sem).wait()
                pltpu.sync_copy(rows, acc.at[db.at[b]], add=True)

    load_blk(0, 0)
    for t0 in range(NRB - 1):
        start(t0, 0, t0)

    def body(i, carry):
        blk0 = 2 * i
        load_blk(blk0 + 1, 1)
        proc(blk0, 0, 1)
        load_blk(blk0 + 2, 0)
        proc(blk0 + 1, 1, 0)
        return carry

    lax.fori_loop(0, NBLK // 2, body, 0)
    plsc.subcore_barrier()
    _striped_copy(s, acc, out_hbm.at[c])


@functools.cache
def _sc_kernels():
    mesh = plsc.VectorSubcoreMesh(
        core_axis_name="c", subcore_axis_name="s",
        num_cores=NC, num_subcores=NS,
    )
    deg = pl.kernel(
        _deg_body,
        out_type=jax.ShapeDtypeStruct((NC, N, DEGW), jnp.float32),
        mesh=mesh,
        scratch_types=[
            pltpu.VMEM((NB, CH), jnp.int32),
            pltpu.VMEM((CH, DEGW), jnp.float32),
            pltpu.VMEM_SHARED((N, DEGW), jnp.float32),
        ],
    )
    spmm = pl.kernel(
        _spmm_body,
        out_type=jax.ShapeDtypeStruct((NC, N, H), jnp.float32),
        mesh=mesh,
        scratch_types=[
            pltpu.VMEM((NB, CH), jnp.int32),
            pltpu.VMEM((NB, CH), jnp.int32),
            pltpu.VMEM((NB, CH), jnp.int32),
            pltpu.VMEM((NB, CH), jnp.int32),
            pltpu.VMEM((CH, H), jnp.float32),
            pltpu.VMEM((CH, H), jnp.float32),
            pltpu.VMEM((CH, H), jnp.float32),
            pltpu.VMEM((CH, H), jnp.float32),
            pltpu.VMEM_SHARED((N, H), jnp.float32),
            pltpu.SemaphoreType.DMA,
            pltpu.SemaphoreType.DMA,
            pltpu.SemaphoreType.DMA,
            pltpu.SemaphoreType.DMA,
        ],
    )
    return deg, spmm


def _tc_in_body(degp_ref, x_ref, w_ref, b_ref, h0_ref, hp0_ref, d_ref):
    deg = degp_ref[0] + degp_ref[1]
    d = lax.rsqrt(jnp.maximum(deg, 1.0))
    d_ref[...] = d
    h0 = jnp.dot(x_ref[...], w_ref[...],
                 preferred_element_type=jnp.float32) + b_ref[...]
    h0_ref[...] = h0
    hp0_ref[...] = h0 * d[:, 0:1]


def _tc_layer_core(aggp_ref, d_ref, hres_ref, w_ref, b_ref, g_ref, be_ref):
    d = d_ref[:, 0:1]
    agg = (aggp_ref[0] + aggp_ref[1]) * d
    t = jnp.dot(agg, w_ref[...],
                preferred_element_type=jnp.float32) + b_ref[...]
    mean = jnp.mean(t, axis=0, keepdims=True)
    ctr = t - mean
    var = jnp.mean(ctr * ctr, axis=0, keepdims=True)
    tn = ctr * lax.rsqrt(var + EPS) * g_ref[...] + be_ref[...]
    h = jnp.maximum(tn, 0.0) + hres_ref[...]
    return h, d


def _tc_layer_body(aggp_ref, d_ref, hres_ref, w_ref, b_ref, g_ref, be_ref,
                   h_ref, hp_ref):
    h, d = _tc_layer_core(aggp_ref, d_ref, hres_ref, w_ref, b_ref, g_ref,
                          be_ref)
    h_ref[...] = h
    hp_ref[...] = h * d


def _tc_last_body(aggp_ref, d_ref, hres_ref, w_ref, b_ref, g_ref, be_ref,
                  wo_ref, bo_ref, out_ref):
    h, _ = _tc_layer_core(aggp_ref, d_ref, hres_ref, w_ref, b_ref, g_ref,
                          be_ref)
    out_ref[...] = jnp.dot(h, wo_ref[...],
                           preferred_element_type=jnp.float32) + bo_ref[...]


_tc_in = pl.pallas_call(
    _tc_in_body,
    out_shape=[
        jax.ShapeDtypeStruct((N, H), jnp.float32),
        jax.ShapeDtypeStruct((N, H), jnp.float32),
        jax.ShapeDtypeStruct((N, DEGW), jnp.float32),
    ],
)

_tc_layer = pl.pallas_call(
    _tc_layer_body,
    out_shape=[
        jax.ShapeDtypeStruct((N, H), jnp.float32),
        jax.ShapeDtypeStruct((N, H), jnp.float32),
    ],
)

_tc_last = pl.pallas_call(
    _tc_last_body,
    out_shape=jax.ShapeDtypeStruct((N, C), jnp.float32),
)


def kernel(x, edge_index, W_in, b_in, W_layers, b_layers, gamma, beta,
           W_out, b_out):
    ei = edge_index.astype(jnp.int32)
    pad = NCHUNK_PAD * CH - E
    src2 = jnp.pad(ei[0], (0, pad)).reshape(NCHUNK_PAD, CH)
    dst2 = jnp.pad(ei[1], (0, pad)).reshape(NCHUNK_PAD, CH)
    zeros_h = jnp.zeros((N, H), jnp.float32)
    zeros_d = jnp.zeros((N, DEGW), jnp.float32)
    ones_d = jnp.ones((CH, DEGW), jnp.float32)

    deg_kernel, spmm_kernel = _sc_kernels()
    degp = deg_kernel(dst2, ones_d, zeros_d)
    h, hp, d = _tc_in(degp, x, W_in, b_in.reshape(1, H))
    for i in range(NLAYERS):
        aggp = spmm_kernel(hp, src2, dst2, zeros_h)
        if i + 1 < NLAYERS:
            h, hp = _tc_layer(
                aggp, d, h, W_layers[i], b_layers[i].reshape(1, H),
                gamma[i].reshape(1, H), beta[i].reshape(1, H)
            )
        else:
            out = _tc_last(
                aggp, d, h, W_layers[i], b_layers[i].reshape(1, H),
                gamma[i].reshape(1, H), beta[i].reshape(1, H),
                W_out, b_out.reshape(1, C)
            )
    return out


# speedup vs baseline: 18.9257x; 1.0131x over previous
import functools

import jax
import jax.numpy as jnp
from jax import lax
from jax.experimental import pallas as pl
from jax.experimental.pallas import tpu as pltpu
from jax.experimental.pallas import tpu_sc as plsc

N = 10000
E = 320000
D = 128
H = 128
C = 40
NLAYERS = 3
EPS = 1e-5

NC = 2
NS = 16
NW = NC * NS
CH = 64
NCHUNK = E // CH
NB = 8
NBLK = 20
CPW = NB * NBLK
NCHUNK_PAD = NW * CPW
DCH = 128
DNCHUNK = E // DCH
DNB = 8
DNBLK = 10
DCPW = DNB * DNBLK
DNCHUNK_PAD = NW * DCPW
RPT = 624
TAIL = N - RPT * NS
TAIL_OFF = RPT * NS
DEGW = 128


def _striped_copy(s, src_ref, dst_ref):
    pltpu.sync_copy(src_ref.at[pl.ds(s * RPT, RPT)],
                    dst_ref.at[pl.ds(s * RPT, RPT)])

    @pl.when(s == NS - 1)
    def _():
        pltpu.sync_copy(src_ref.at[pl.ds(TAIL_OFF, TAIL)],
                        dst_ref.at[pl.ds(TAIL_OFF, TAIL)])


def _deg_body(dst2_hbm, ones_hbm, zeros_hbm, out_hbm, dblk, ones_v, acc):
    c = lax.axis_index("c")
    s = lax.axis_index("s")
    w = s * NC + c
    base = w * DCPW
    pltpu.sync_copy(ones_hbm, ones_v)
    _striped_copy(s, zeros_hbm, acc)
    plsc.subcore_barrier()

    def blk_body(blk, carry):
        off = base + blk * DNB

        @pl.when(off < DNCHUNK)
        def _():
            pltpu.sync_copy(dst2_hbm.at[pl.ds(off, DNB)], dblk)
            for b in range(DNB):
                @pl.when(off + b < DNCHUNK)
                def _b():
                    pltpu.sync_copy(ones_v, acc.at[dblk.at[b]], add=True)

        return carry

    lax.fori_loop(0, DNBLK, blk_body, 0)
    plsc.subcore_barrier()
    _striped_copy(s, acc, out_hbm.at[c])


NRB = 4


def _spmm_body(hp_hbm, src2_hbm, dst2_hbm, zeros_hbm, out_hbm,
               sblk0, dblk0, sblk1, dblk1,
               rows0, rows1, rows2, rows3, acc, sem0, sem1, sem2, sem3):
    c = lax.axis_index("c")
    s = lax.axis_index("s")
    w = s * NC + c
    base = w * CPW
    _striped_copy(s, zeros_hbm, acc)
    plsc.subcore_barrier()

    iblk = ((sblk0, dblk0), (sblk1, dblk1))
    rbuf = ((rows0, sem0), (rows1, sem1), (rows2, sem2), (rows3, sem3))

    def load_blk(blk, p):
        sb, db = iblk[p]
        off = base + blk * NB

        @pl.when(off < NCHUNK)
        def _():
            pltpu.sync_copy(src2_hbm.at[pl.ds(off, NB)], sb)
            pltpu.sync_copy(dst2_hbm.at[pl.ds(off, NB)], db)

    def start(t, p, b):
        sb, _ = iblk[p]
        rows, sem = rbuf[b % NRB]

        @pl.when((base + t < NCHUNK) & (t < CPW))
        def _():
            pltpu.async_copy(hp_hbm.at[sb.at[b]], rows, sem)

    def proc(blk, p, pn):
        sb, db = iblk[p]
        for b in range(NB):
            t = blk * NB + b
            bn = b + NRB - 1
            if bn < NB:
                start(t + NRB - 1, p, bn)
            else:
                start(t + NRB - 1, pn, bn - NB)
            rows, sem = rbuf[b % NRB]

            @pl.when(base + t < NCHUNK)
            def _():
                pltpu.make_async_copy(hp_hbm.at[sb.at[b]], rows, sem).wait()
                pltpu.sync_copy(rows, acc.at[db.at[b]], add=True)

    load_blk(0, 0)
    for t0 in range(NRB - 1):
        start(t0, 0, t0)

    def body(i, carry):
        blk0 = 2 * i
        load_blk(blk0 + 1, 1)
        proc(blk0, 0, 1)
        load_blk(blk0 + 2, 0)
        proc(blk0 + 1, 1, 0)
        return carry

    lax.fori_loop(0, NBLK // 2, body, 0)
    plsc.subcore_barrier()
    _striped_copy(s, acc, out_hbm.at[c])


@functools.cache
def _sc_kernels():
    mesh = plsc.VectorSubcoreMesh(
        core_axis_name="c", subcore_axis_name="s",
        num_cores=NC, num_subcores=NS,
    )
    deg = pl.kernel(
        _deg_body,
        out_type=jax.ShapeDtypeStruct((NC, N, DEGW), jnp.float32),
        mesh=mesh,
        scratch_types=[
            pltpu.VMEM((DNB, DCH), jnp.int32),
            pltpu.VMEM((DCH, DEGW), jnp.float32),
            pltpu.VMEM_SHARED((N, DEGW), jnp.float32),
        ],
    )
    spmm = pl.kernel(
        _spmm_body,
        out_type=jax.ShapeDtypeStruct((NC, N, H), jnp.float32),
        mesh=mesh,
        scratch_types=[
            pltpu.VMEM((NB, CH), jnp.int32),
            pltpu.VMEM((NB, CH), jnp.int32),
            pltpu.VMEM((NB, CH), jnp.int32),
            pltpu.VMEM((NB, CH), jnp.int32),
            pltpu.VMEM((CH, H), jnp.float32),
            pltpu.VMEM((CH, H), jnp.float32),
            pltpu.VMEM((CH, H), jnp.float32),
            pltpu.VMEM((CH, H), jnp.float32),
            pltpu.VMEM_SHARED((N, H), jnp.float32),
            pltpu.SemaphoreType.DMA,
            pltpu.SemaphoreType.DMA,
            pltpu.SemaphoreType.DMA,
            pltpu.SemaphoreType.DMA,
        ],
    )
    return deg, spmm


def _tc_in_body(degp_ref, x_ref, w_ref, b_ref, h0_ref, hp0_ref, d_ref):
    deg = degp_ref[0] + degp_ref[1]
    d = lax.rsqrt(jnp.maximum(deg, 1.0))
    d_ref[...] = d[:, 0:8]
    h0 = jnp.dot(x_ref[...], w_ref[...],
                 preferred_element_type=jnp.float32) + b_ref[...]
    h0_ref[...] = h0
    hp0_ref[...] = h0 * d[:, 0:1]


def _tc_layer_core(aggp_ref, d_ref, hres_ref, w_ref, b_ref, g_ref, be_ref):
    d = d_ref[:, 0:1]
    agg = (aggp_ref[0] + aggp_ref[1]) * d
    t = jnp.dot(agg, w_ref[...],
                preferred_element_type=jnp.float32) + b_ref[...]
    mean = jnp.mean(t, axis=0, keepdims=True)
    ctr = t - mean
    var = jnp.mean(ctr * ctr, axis=0, keepdims=True)
    tn = ctr * lax.rsqrt(var + EPS) * g_ref[...] + be_ref[...]
    h = jnp.maximum(tn, 0.0) + hres_ref[...]
    return h, d


def _tc_layer_body(aggp_ref, d_ref, hres_ref, w_ref, b_ref, g_ref, be_ref,
                   h_ref, hp_ref):
    h, d = _tc_layer_core(aggp_ref, d_ref, hres_ref, w_ref, b_ref, g_ref,
                          be_ref)
    h_ref[...] = h
    hp_ref[...] = h * d


def _tc_last_body(aggp_ref, d_ref, hres_ref, w_ref, b_ref, g_ref, be_ref,
                  wo_ref, bo_ref, out_ref):
    h, _ = _tc_layer_core(aggp_ref, d_ref, hres_ref, w_ref, b_ref, g_ref,
                          be_ref)
    out_ref[...] = jnp.dot(h, wo_ref[...],
                           preferred_element_type=jnp.float32) + bo_ref[...]


_tc_in = pl.pallas_call(
    _tc_in_body,
    out_shape=[
        jax.ShapeDtypeStruct((N, H), jnp.float32),
        jax.ShapeDtypeStruct((N, H), jnp.float32),
        jax.ShapeDtypeStruct((N, 8), jnp.float32),
    ],
)

_tc_layer = pl.pallas_call(
    _tc_layer_body,
    out_shape=[
        jax.ShapeDtypeStruct((N, H), jnp.float32),
        jax.ShapeDtypeStruct((N, H), jnp.float32),
    ],
)

_tc_last = pl.pallas_call(
    _tc_last_body,
    out_shape=jax.ShapeDtypeStruct((N, C), jnp.float32),
)


def kernel(x, edge_index, W_in, b_in, W_layers, b_layers, gamma, beta,
           W_out, b_out):
    ei = edge_index.astype(jnp.int32)
    pad = NCHUNK_PAD * CH - E
    src2 = jnp.pad(ei[0], (0, pad)).reshape(NCHUNK_PAD, CH)
    dst_flat = jnp.pad(ei[1], (0, pad))
    dst2 = dst_flat.reshape(NCHUNK_PAD, CH)
    dst3 = dst_flat.reshape(DNCHUNK_PAD, DCH)
    zeros_h = jnp.zeros((N, H), jnp.float32)
    zeros_d = jnp.zeros((N, DEGW), jnp.float32)
    ones_d = jnp.ones((DCH, DEGW), jnp.float32)

    deg_kernel, spmm_kernel = _sc_kernels()
    degp = deg_kernel(dst3, ones_d, zeros_d)
    h, hp, d = _tc_in(degp, x, W_in, b_in.reshape(1, H))
    for i in range(NLAYERS):
        aggp = spmm_kernel(hp, src2, dst2, zeros_h)
        if i + 1 < NLAYERS:
            h, hp = _tc_layer(
                aggp, d, h, W_layers[i], b_layers[i].reshape(1, H),
                gamma[i].reshape(1, H), beta[i].reshape(1, H)
            )
        else:
            out = _tc_last(
                aggp, d, h, W_layers[i], b_layers[i].reshape(1, H),
                gamma[i].reshape(1, H), beta[i].reshape(1, H),
                W_out, b_out.reshape(1, C)
            )
    return out
